# baseline (speedup 1.0000x reference)
import functools

import jax
import jax.numpy as jnp
from jax import lax
from jax.experimental import pallas as pl
from jax.experimental.pallas import tpu as pltpu
from jax.experimental.pallas import tpu_sc as plsc

N = 2048
D = 768
H = 768
E = 8
C = 512
SENT = E * C
ROWS = 9 * C

BLK = 256
NB = N // BLK

NC, NS = 2, 16
NW = NC * NS
TPW = N // NW


def _k1_body(x0_ref, s_ref, b_ref, wr_ref, br_ref,
             xact_ref, d0_ref, d1_ref, g0_ref, g1_ref, carry_ref):
    blk = pl.program_id(0)
    x = x0_ref[...]
    mean = jnp.mean(x, axis=-1, keepdims=True)
    var = jnp.mean((x - mean) ** 2, axis=-1, keepdims=True)
    xn = (x - mean) * lax.rsqrt(var + 1e-6) * s_ref[...] + b_ref[...]
    xa = jnp.maximum(xn, 0.0)
    xact_ref[...] = xa

    logits = jnp.dot(xa, wr_ref[...], preferred_element_type=jnp.float32)
    logits = logits + br_ref[...]

    eidx = lax.broadcasted_iota(jnp.int32, (BLK, E), 1)
    m0 = jnp.max(logits, axis=-1, keepdims=True)
    i0 = jnp.min(jnp.where(logits == m0, eidx, E), axis=-1, keepdims=True)
    neg = jnp.where(eidx == i0, -jnp.inf, logits)
    m1 = jnp.max(neg, axis=-1, keepdims=True)
    i1 = jnp.min(jnp.where(neg == m1, eidx, E), axis=-1, keepdims=True)

    e1 = jnp.exp(m1 - m0)
    g0_ref[...] = 1.0 / (1.0 + e1)
    g1_ref[...] = e1 / (1.0 + e1)

    oh0 = (eidx == i0).astype(jnp.float32)
    oh1 = (eidx == i1).astype(jnp.float32)
    ohs = oh0 + oh1

    r = lax.broadcasted_iota(jnp.int32, (BLK, BLK), 0)
    c = lax.broadcasted_iota(jnp.int32, (BLK, BLK), 1)
    tri = (c < r).astype(jnp.float32)
    cumb = jnp.dot(tri, ohs, preferred_element_type=jnp.float32)

    @pl.when(blk == 0)
    def _():
        carry_ref[...] = jnp.zeros((1, E), jnp.float32)

    base = cumb + carry_ref[...]
    carry_ref[...] = carry_ref[...] + jnp.sum(ohs, axis=0, keepdims=True)

    p0 = jnp.sum(base * oh0, axis=-1, keepdims=True).astype(jnp.int32)
    p1 = jnp.sum(base * oh1, axis=-1, keepdims=True).astype(jnp.int32)
    d0_ref[...] = jnp.where(p0 < C, i0 * C + p0, SENT)
    d1_ref[...] = jnp.where(p1 < C, i1 * C + p1, SENT)


def _k1_call(x0f, ln0_scale, ln0_bias, Wr, br):
    f32 = jnp.float32
    return pl.pallas_call(
        _k1_body,
        grid=(NB,),
        in_specs=[
            pl.BlockSpec((BLK, D), lambda b: (b, 0)),
            pl.BlockSpec((1, D), lambda b: (0, 0)),
            pl.BlockSpec((1, D), lambda b: (0, 0)),
            pl.BlockSpec((D, E), lambda b: (0, 0)),
            pl.BlockSpec((1, E), lambda b: (0, 0)),
        ],
        out_specs=[
            pl.BlockSpec((BLK, D), lambda b: (b, 0)),
            pl.BlockSpec((BLK, 1), lambda b: (b, 0)),
            pl.BlockSpec((BLK, 1), lambda b: (b, 0)),
            pl.BlockSpec((BLK, 1), lambda b: (b, 0)),
            pl.BlockSpec((BLK, 1), lambda b: (b, 0)),
        ],
        out_shape=[
            jax.ShapeDtypeStruct((N, D), f32),
            jax.ShapeDtypeStruct((N, 1), jnp.int32),
            jax.ShapeDtypeStruct((N, 1), jnp.int32),
            jax.ShapeDtypeStruct((N, 1), f32),
            jax.ShapeDtypeStruct((N, 1), f32),
        ],
        scratch_shapes=[pltpu.VMEM((1, E), f32)],
    )(x0f, ln0_scale.reshape(1, D), ln0_bias.reshape(1, D),
      Wr, br.reshape(1, E))


def _k1g_body(d0_ref, d1_ref, g0_ref, g1_ref, gsl_ref):
    e = pl.program_id(0)
    slot = e * C + lax.broadcasted_iota(jnp.int32, (C, 1), 0)
    eq0 = slot == d0_ref[...]
    eq1 = slot == d1_ref[...]
    s0 = jnp.sum(jnp.where(eq0, g0_ref[...], 0.0), axis=1, keepdims=True)
    s1 = jnp.sum(jnp.where(eq1, g1_ref[...], 0.0), axis=1, keepdims=True)
    gsl_ref[...] = s0 + s1


def _k1g_call(d0r, d1r, g0r, g1r):
    row = pl.BlockSpec((1, N), lambda e: (0, 0))
    return pl.pallas_call(
        _k1g_body,
        grid=(E,),
        in_specs=[row, row, row, row],
        out_specs=pl.BlockSpec((C, 1), lambda e: (e, 0)),
        out_shape=jax.ShapeDtypeStruct((ROWS, 1), jnp.float32),
    )(d0r, d1r, g0r, g1r)


def _k2_body(x_hbm, d0_hbm, d1_hbm, xe_hbm, idx0_v, idx1_v, rows_v, sem):
    wid = lax.axis_index("s") * NC + lax.axis_index("c")
    base = wid * TPW
    pltpu.sync_copy(d0_hbm.at[pl.ds(base, TPW)], idx0_v)
    pltpu.sync_copy(d1_hbm.at[pl.ds(base, TPW)], idx1_v)
    pltpu.sync_copy(x_hbm.at[pl.ds(base, TPW)], rows_v)
    pltpu.async_copy(rows_v, xe_hbm.at[idx0_v], sem).wait()
    pltpu.async_copy(rows_v, xe_hbm.at[idx1_v], sem).wait()


@functools.cache
def _k2_kernel():
    return pl.kernel(
        _k2_body,
        mesh=plsc.VectorSubcoreMesh(core_axis_name="c", subcore_axis_name="s"),
        out_type=jax.ShapeDtypeStruct((ROWS, D), jnp.float32),
        scratch_types=[
            pltpu.VMEM((TPW,), jnp.int32),
            pltpu.VMEM((TPW,), jnp.int32),
            pltpu.VMEM((TPW, D), jnp.float32),
            pltpu.SemaphoreType.DMA,
        ],
    )


def _k2_call(xact, d0f, d1f):
    return _k2_kernel()(xact, d0f, d1f)


def _k3_body(xe_ref, gsl_ref, w1_ref, b1_ref, s1_ref, bb1_ref, w2_ref, b2_ref,
             y_ref):
    e = pl.program_id(0)

    @pl.when(e < E)
    def _():
        x = xe_ref[...].astype(jnp.bfloat16)
        h = jnp.dot(x, w1_ref[0], preferred_element_type=jnp.float32)
        h = h + b1_ref[0]
        mean = jnp.mean(h, axis=-1, keepdims=True)
        var = jnp.mean((h - mean) ** 2, axis=-1, keepdims=True)
        h = (h - mean) * lax.rsqrt(var + 1e-6) * s1_ref[0] + bb1_ref[0]
        h = jnp.maximum(h, 0.0).astype(jnp.bfloat16)
        y = jnp.dot(h, w2_ref[0], preferred_element_type=jnp.float32)
        y_ref[...] = (y + b2_ref[0]) * gsl_ref[...]

    @pl.when(e == E)
    def _():
        y_ref[...] = jnp.zeros((C, D), jnp.float32)


def _k3_call(xe, gsl, W1b, b1, ln1_scale, ln1_bias, W2b, b2):
    vec = pl.BlockSpec((1, 1, H), lambda e: (jnp.minimum(e, E - 1), 0, 0))
    return pl.pallas_call(
        _k3_body,
        grid=(E + 1,),
        in_specs=[
            pl.BlockSpec((C, D), lambda e: (e, 0)),
            pl.BlockSpec((C, 1), lambda e: (e, 0)),
            pl.BlockSpec((1, D, H), lambda e: (jnp.minimum(e, E - 1), 0, 0)),
            vec, vec, vec,
            pl.BlockSpec((1, H, D), lambda e: (jnp.minimum(e, E - 1), 0, 0)),
            pl.BlockSpec((1, 1, D), lambda e: (jnp.minimum(e, E - 1), 0, 0)),
        ],
        out_specs=pl.BlockSpec((C, D), lambda e: (e, 0)),
        out_shape=jax.ShapeDtypeStruct((ROWS, D), jnp.float32),
    )(xe, gsl, W1b, b1.reshape(E, 1, H), ln1_scale.reshape(E, 1, H),
      ln1_bias.reshape(E, 1, H), W2b, b2.reshape(E, 1, D))


def _k4_body(y_hbm, d0_hbm, d1_hbm, ys_hbm, i0_v, i1_v, r0_v, r1_v, sem):
    wid = lax.axis_index("s") * NC + lax.axis_index("c")
    base = wid * TPW
    pltpu.sync_copy(d0_hbm.at[pl.ds(base, TPW)], i0_v)
    pltpu.sync_copy(d1_hbm.at[pl.ds(base, TPW)], i1_v)
    pltpu.async_copy(y_hbm.at[i0_v], r0_v, sem).wait()
    pltpu.async_copy(y_hbm.at[i1_v], r1_v, sem).wait()

    def tok(i, _):
        for j in range(D // 16):
            sl = pl.ds(j * 16, 16)
            r0_v[i, sl] = r0_v[i, sl] + r1_v[i, sl]
        return 0

    lax.fori_loop(0, TPW, tok, 0)
    pltpu.sync_copy(r0_v, ys_hbm.at[pl.ds(base, TPW)])


@functools.cache
def _k4_kernel():
    return pl.kernel(
        _k4_body,
        mesh=plsc.VectorSubcoreMesh(core_axis_name="c", subcore_axis_name="s"),
        out_type=jax.ShapeDtypeStruct((N, D), jnp.float32),
        scratch_types=[
            pltpu.VMEM((TPW,), jnp.int32),
            pltpu.VMEM((TPW,), jnp.int32),
            pltpu.VMEM((TPW, D), jnp.float32),
            pltpu.VMEM((TPW, D), jnp.float32),
            pltpu.SemaphoreType.DMA,
        ],
    )


def _k4_call(ybuf, d0f, d1f):
    return _k4_kernel()(ybuf, d0f, d1f)


def _k5_body(x0_ref, ys_ref, o_ref):
    o_ref[...] = x0_ref[...] + ys_ref[...]


def _k5_call(x0f, ysum):
    row = pl.BlockSpec((BLK, D), lambda b: (b, 0))
    return pl.pallas_call(
        _k5_body,
        grid=(NB,),
        in_specs=[row, row],
        out_specs=row,
        out_shape=jax.ShapeDtypeStruct((N, D), jnp.float32),
    )(x0f, ysum)


def kernel(x0, ln0_scale, ln0_bias, Wr, br, W1, b1, ln1_scale, ln1_bias, W2, b2):
    x0f = x0.reshape(N, D)
    xact, d0, d1, g0, g1 = _k1_call(x0f, ln0_scale, ln0_bias, Wr, br)
    d0f = d0.reshape(N)
    d1f = d1.reshape(N)
    gsl = _k1g_call(d0.reshape(1, N), d1.reshape(1, N),
                    g0.reshape(1, N), g1.reshape(1, N))
    xe = _k2_call(xact, d0f, d1f)
    ybuf = _k3_call(xe, gsl, W1.astype(jnp.bfloat16), b1, ln1_scale, ln1_bias,
                    W2.astype(jnp.bfloat16), b2)
    ysum = _k4_call(ybuf, d0f, d1f)
    out = _k5_call(x0f, ysum)
    return out.reshape(x0.shape)


# speedup vs baseline: 1.2980x; 1.2980x over previous
import functools

import jax
import jax.numpy as jnp
from jax import lax
from jax.experimental import pallas as pl
from jax.experimental.pallas import tpu as pltpu
from jax.experimental.pallas import tpu_sc as plsc

N = 2048
D = 768
H = 768
E = 8
C = 512
SENT = E * C
ROWS = 9 * C

BLK = 256
NB = N // BLK

NC, NS = 2, 16
NW = NC * NS
TPW = N // NW
NSC = 1024
TPW4 = NSC // NW
BLK4 = 512


def _k1_body(x0_ref, s_ref, b_ref, wr_ref, br_ref,
             xact_ref, d0_ref, d1_ref, g0_ref, g1_ref, carry_ref):
    blk = pl.program_id(0)
    x = x0_ref[...]
    mean = jnp.mean(x, axis=-1, keepdims=True)
    var = jnp.mean((x - mean) ** 2, axis=-1, keepdims=True)
    xn = (x - mean) * lax.rsqrt(var + 1e-6) * s_ref[...] + b_ref[...]
    xa = jnp.maximum(xn, 0.0)
    xact_ref[...] = xa

    logits = jnp.dot(xa, wr_ref[...], preferred_element_type=jnp.float32)
    logits = logits + br_ref[...]

    eidx = lax.broadcasted_iota(jnp.int32, (BLK, E), 1)
    m0 = jnp.max(logits, axis=-1, keepdims=True)
    i0 = jnp.min(jnp.where(logits == m0, eidx, E), axis=-1, keepdims=True)
    neg = jnp.where(eidx == i0, -jnp.inf, logits)
    m1 = jnp.max(neg, axis=-1, keepdims=True)
    i1 = jnp.min(jnp.where(neg == m1, eidx, E), axis=-1, keepdims=True)

    e1 = jnp.exp(m1 - m0)
    g0_ref[...] = 1.0 / (1.0 + e1)
    g1_ref[...] = e1 / (1.0 + e1)

    oh0 = (eidx == i0).astype(jnp.float32)
    oh1 = (eidx == i1).astype(jnp.float32)
    ohs = oh0 + oh1

    r = lax.broadcasted_iota(jnp.int32, (BLK, BLK), 0)
    c = lax.broadcasted_iota(jnp.int32, (BLK, BLK), 1)
    tri = (c < r).astype(jnp.float32)
    cumb = jnp.dot(tri, ohs, preferred_element_type=jnp.float32)

    @pl.when(blk == 0)
    def _():
        carry_ref[...] = jnp.zeros((1, E), jnp.float32)

    base = cumb + carry_ref[...]
    carry_ref[...] = carry_ref[...] + jnp.sum(ohs, axis=0, keepdims=True)

    p0 = jnp.sum(base * oh0, axis=-1, keepdims=True).astype(jnp.int32)
    p1 = jnp.sum(base * oh1, axis=-1, keepdims=True).astype(jnp.int32)
    d0_ref[...] = jnp.where(p0 < C, i0 * C + p0, SENT)
    d1_ref[...] = jnp.where(p1 < C, i1 * C + p1, SENT)


def _k1_call(x0f, ln0_scale, ln0_bias, Wr, br):
    f32 = jnp.float32
    return pl.pallas_call(
        _k1_body,
        grid=(NB,),
        in_specs=[
            pl.BlockSpec((BLK, D), lambda b: (b, 0)),
            pl.BlockSpec((1, D), lambda b: (0, 0)),
            pl.BlockSpec((1, D), lambda b: (0, 0)),
            pl.BlockSpec((D, E), lambda b: (0, 0)),
            pl.BlockSpec((1, E), lambda b: (0, 0)),
        ],
        out_specs=[
            pl.BlockSpec((BLK, D), lambda b: (b, 0)),
            pl.BlockSpec((BLK, 1), lambda b: (b, 0)),
            pl.BlockSpec((BLK, 1), lambda b: (b, 0)),
            pl.BlockSpec((BLK, 1), lambda b: (b, 0)),
            pl.BlockSpec((BLK, 1), lambda b: (b, 0)),
        ],
        out_shape=[
            jax.ShapeDtypeStruct((N, D), f32),
            jax.ShapeDtypeStruct((N, 1), jnp.int32),
            jax.ShapeDtypeStruct((N, 1), jnp.int32),
            jax.ShapeDtypeStruct((N, 1), f32),
            jax.ShapeDtypeStruct((N, 1), f32),
        ],
        scratch_shapes=[pltpu.VMEM((1, E), f32)],
    )(x0f, ln0_scale.reshape(1, D), ln0_bias.reshape(1, D),
      Wr, br.reshape(1, E))


def _k1g_body(d0_ref, d1_ref, g0_ref, g1_ref, gsl_ref):
    e = pl.program_id(0)
    slot = e * C + lax.broadcasted_iota(jnp.int32, (C, 1), 0)
    eq0 = slot == d0_ref[...]
    eq1 = slot == d1_ref[...]
    s0 = jnp.sum(jnp.where(eq0, g0_ref[...], 0.0), axis=1, keepdims=True)
    s1 = jnp.sum(jnp.where(eq1, g1_ref[...], 0.0), axis=1, keepdims=True)
    gsl_ref[...] = s0 + s1


def _k1g_call(d0r, d1r, g0r, g1r):
    row = pl.BlockSpec((1, N), lambda e: (0, 0))
    return pl.pallas_call(
        _k1g_body,
        grid=(E,),
        in_specs=[row, row, row, row],
        out_specs=pl.BlockSpec((C, 1), lambda e: (e, 0)),
        out_shape=jax.ShapeDtypeStruct((ROWS, 1), jnp.float32),
    )(d0r, d1r, g0r, g1r)


def _k2_body(x_hbm, d0_hbm, d1_hbm, xe_hbm, idx0_v, idx1_v, rows_v, sem):
    wid = lax.axis_index("s") * NC + lax.axis_index("c")
    base = wid * TPW
    pltpu.sync_copy(d0_hbm.at[pl.ds(base, TPW)], idx0_v)
    pltpu.sync_copy(d1_hbm.at[pl.ds(base, TPW)], idx1_v)
    pltpu.sync_copy(x_hbm.at[pl.ds(base, TPW)], rows_v)
    pltpu.async_copy(rows_v, xe_hbm.at[idx0_v], sem).wait()
    pltpu.async_copy(rows_v, xe_hbm.at[idx1_v], sem).wait()


@functools.cache
def _k2_kernel():
    return pl.kernel(
        _k2_body,
        mesh=plsc.VectorSubcoreMesh(core_axis_name="c", subcore_axis_name="s"),
        out_type=jax.ShapeDtypeStruct((ROWS, D), jnp.float32),
        scratch_types=[
            pltpu.VMEM((TPW,), jnp.int32),
            pltpu.VMEM((TPW,), jnp.int32),
            pltpu.VMEM((TPW, D), jnp.float32),
            pltpu.SemaphoreType.DMA,
        ],
    )


def _k2_call(xact, d0f, d1f):
    return _k2_kernel()(xact, d0f, d1f)


def _k3_body(xe_ref, gsl_ref, w1_ref, b1_ref, s1_ref, bb1_ref, w2_ref, b2_ref,
             y_ref, yb_ref):
    e = pl.program_id(0)

    @pl.when(e < E)
    def _():
        x = xe_ref[...].astype(jnp.bfloat16)
        h = jnp.dot(x, w1_ref[0], preferred_element_type=jnp.float32)
        h = h + b1_ref[0]
        mean = jnp.mean(h, axis=-1, keepdims=True)
        var = jnp.mean((h - mean) ** 2, axis=-1, keepdims=True)
        h = (h - mean) * lax.rsqrt(var + 1e-6) * s1_ref[0] + bb1_ref[0]
        h = jnp.maximum(h, 0.0).astype(jnp.bfloat16)
        y = jnp.dot(h, w2_ref[0], preferred_element_type=jnp.float32)
        g = gsl_ref[...]
        y = (y + b2_ref[0]) * g
        y_ref[...] = y
        yb_ref[...] = jnp.where(g > 0, y, 0.0).astype(jnp.bfloat16)

    @pl.when(e == E)
    def _():
        y_ref[...] = jnp.zeros((C, D), jnp.float32)
        yb_ref[...] = jnp.zeros((C, D), jnp.bfloat16)


def _k3_call(xe, gsl, W1b, b1, ln1_scale, ln1_bias, W2b, b2):
    vec = pl.BlockSpec((1, 1, H), lambda e: (jnp.minimum(e, E - 1), 0, 0))
    return pl.pallas_call(
        _k3_body,
        grid=(E + 1,),
        in_specs=[
            pl.BlockSpec((C, D), lambda e: (e, 0)),
            pl.BlockSpec((C, 1), lambda e: (e, 0)),
            pl.BlockSpec((1, D, H), lambda e: (jnp.minimum(e, E - 1), 0, 0)),
            vec, vec, vec,
            pl.BlockSpec((1, H, D), lambda e: (jnp.minimum(e, E - 1), 0, 0)),
            pl.BlockSpec((1, 1, D), lambda e: (jnp.minimum(e, E - 1), 0, 0)),
        ],
        out_specs=[pl.BlockSpec((C, D), lambda e: (e, 0)),
                   pl.BlockSpec((C, D), lambda e: (e, 0))],
        out_shape=[jax.ShapeDtypeStruct((ROWS, D), jnp.float32),
                   jax.ShapeDtypeStruct((ROWS, D), jnp.bfloat16)],
    )(xe, gsl, W1b, b1.reshape(E, 1, H), ln1_scale.reshape(E, 1, H),
      ln1_bias.reshape(E, 1, H), W2b, b2.reshape(E, 1, D))


def _k4_body(y_hbm, d0_hbm, d1_hbm, ys_hbm, i0_v, i1_v, r0_v, r1_v, sem):
    wid = lax.axis_index("s") * NC + lax.axis_index("c")
    base = wid * TPW4
    pltpu.sync_copy(d0_hbm.at[pl.ds(base, TPW4)], i0_v)
    pltpu.sync_copy(d1_hbm.at[pl.ds(base, TPW4)], i1_v)
    pltpu.async_copy(y_hbm.at[i0_v], r0_v, sem).wait()
    pltpu.async_copy(y_hbm.at[i1_v], r1_v, sem).wait()

    def tok(i, _):
        for j in range(D // 16):
            sl = pl.ds(j * 16, 16)
            r0_v[i, sl] = r0_v[i, sl] + r1_v[i, sl]
        return 0

    lax.fori_loop(0, TPW4, tok, 0)
    pltpu.sync_copy(r0_v, ys_hbm.at[pl.ds(base, TPW4)])


@functools.cache
def _k4_kernel():
    return pl.kernel(
        _k4_body,
        mesh=plsc.VectorSubcoreMesh(core_axis_name="c", subcore_axis_name="s"),
        out_type=jax.ShapeDtypeStruct((NSC, D), jnp.float32),
        scratch_types=[
            pltpu.VMEM((TPW4,), jnp.int32),
            pltpu.VMEM((TPW4,), jnp.int32),
            pltpu.VMEM((TPW4, D), jnp.float32),
            pltpu.VMEM((TPW4, D), jnp.float32),
            pltpu.SemaphoreType.DMA,
        ],
    )


def _k4_call(ybuf, d0f, d1f):
    return _k4_kernel()(ybuf, d0f, d1f)


def _k4t_body(d0_ref, d1_ref, x0_ref, yb_ref, o_ref):
    slot = lax.broadcasted_iota(jnp.int32, (BLK4, ROWS), 1)
    comb = ((slot == d0_ref[...]) | (slot == d1_ref[...])).astype(jnp.bfloat16)
    mix = jnp.dot(comb, yb_ref[...], preferred_element_type=jnp.float32)
    o_ref[...] = x0_ref[...] + mix


def _k4t_call(d0, d1, x0f, ybf):
    off = NSC // BLK4
    col = pl.BlockSpec((BLK4, 1), lambda b: (off + b, 0))
    return pl.pallas_call(
        _k4t_body,
        grid=((N - NSC) // BLK4,),
        in_specs=[
            col, col,
            pl.BlockSpec((BLK4, D), lambda b: (off + b, 0)),
            pl.BlockSpec((ROWS, D), lambda b: (0, 0)),
        ],
        out_specs=pl.BlockSpec((BLK4, D), lambda b: (b, 0)),
        out_shape=jax.ShapeDtypeStruct((N - NSC, D), jnp.float32),
    )(d0, d1, x0f, ybf)


def _k5_body(x0_ref, ys_ref, yt_ref, o_ref):
    b = pl.program_id(0)

    @pl.when(b < NSC // BLK)
    def _():
        o_ref[...] = x0_ref[...] + ys_ref[...]

    @pl.when(b >= NSC // BLK)
    def _():
        o_ref[...] = yt_ref[...]


def _k5_call(x0f, ysum, ytc):
    nlo = NSC // BLK
    return pl.pallas_call(
        _k5_body,
        grid=(NB,),
        in_specs=[
            pl.BlockSpec((BLK, D), lambda b: (b, 0)),
            pl.BlockSpec((BLK, D), lambda b: (jnp.minimum(b, nlo - 1), 0)),
            pl.BlockSpec((BLK, D), lambda b: (jnp.maximum(b - nlo, 0), 0)),
        ],
        out_specs=pl.BlockSpec((BLK, D), lambda b: (b, 0)),
        out_shape=jax.ShapeDtypeStruct((N, D), jnp.float32),
    )(x0f, ysum, ytc)


def kernel(x0, ln0_scale, ln0_bias, Wr, br, W1, b1, ln1_scale, ln1_bias, W2, b2):
    x0f = x0.reshape(N, D)
    xact, d0, d1, g0, g1 = _k1_call(x0f, ln0_scale, ln0_bias, Wr, br)
    d0f = d0.reshape(N)
    d1f = d1.reshape(N)
    gsl = _k1g_call(d0.reshape(1, N), d1.reshape(1, N),
                    g0.reshape(1, N), g1.reshape(1, N))
    xe = _k2_call(xact, d0f, d1f)
    ybuf, ybf = _k3_call(xe, gsl, W1.astype(jnp.bfloat16), b1, ln1_scale,
                         ln1_bias, W2.astype(jnp.bfloat16), b2)
    ysum = _k4_call(ybuf, d0f, d1f)
    ytc = _k4t_call(d0, d1, x0f, ybf)
    out = _k5_call(x0f, ysum, ytc)
    return out.reshape(x0.shape)


# speedup vs baseline: 1.5473x; 1.1921x over previous
import functools

import jax
import jax.numpy as jnp
from jax import lax
from jax.experimental import pallas as pl
from jax.experimental.pallas import tpu as pltpu
from jax.experimental.pallas import tpu_sc as plsc

N = 2048
D = 768
H = 768
E = 8
C = 512
SENT = E * C
ROWS = 9 * C

BLK = 256
NB = N // BLK

NC, NS = 2, 16
NW = NC * NS
TPW = N // NW
NSC = 1024
TPW4 = NSC // NW
BLK4 = 512


def _k1_body(x0_ref, s_ref, b_ref, wr_ref, br_ref,
             xlo_ref, xup_ref, d0_ref, d1_ref, g0_ref, g1_ref, carry_ref):
    blk = pl.program_id(0)
    x = x0_ref[...]
    mean = jnp.mean(x, axis=-1, keepdims=True)
    var = jnp.mean((x - mean) ** 2, axis=-1, keepdims=True)
    xn = (x - mean) * lax.rsqrt(var + 1e-6) * s_ref[...] + b_ref[...]
    xa = jnp.maximum(xn, 0.0)

    @pl.when(blk < NSC // BLK)
    def _():
        xlo_ref[...] = xa

    @pl.when(blk >= NSC // BLK)
    def _():
        xup_ref[...] = xa.astype(jnp.bfloat16)

    logits = jnp.dot(xa, wr_ref[...], preferred_element_type=jnp.float32)
    logits = logits + br_ref[...]

    eidx = lax.broadcasted_iota(jnp.int32, (BLK, E), 1)
    m0 = jnp.max(logits, axis=-1, keepdims=True)
    i0 = jnp.min(jnp.where(logits == m0, eidx, E), axis=-1, keepdims=True)
    neg = jnp.where(eidx == i0, -jnp.inf, logits)
    m1 = jnp.max(neg, axis=-1, keepdims=True)
    i1 = jnp.min(jnp.where(neg == m1, eidx, E), axis=-1, keepdims=True)

    e1 = jnp.exp(m1 - m0)
    g0_ref[...] = 1.0 / (1.0 + e1)
    g1_ref[...] = e1 / (1.0 + e1)

    oh0 = (eidx == i0).astype(jnp.float32)
    oh1 = (eidx == i1).astype(jnp.float32)
    ohs = oh0 + oh1

    r = lax.broadcasted_iota(jnp.int32, (BLK, BLK), 0)
    c = lax.broadcasted_iota(jnp.int32, (BLK, BLK), 1)
    tri = (c < r).astype(jnp.float32)
    cumb = jnp.dot(tri, ohs, preferred_element_type=jnp.float32)

    @pl.when(blk == 0)
    def _():
        carry_ref[...] = jnp.zeros((1, E), jnp.float32)

    base = cumb + carry_ref[...]
    carry_ref[...] = carry_ref[...] + jnp.sum(ohs, axis=0, keepdims=True)

    p0 = jnp.sum(base * oh0, axis=-1, keepdims=True).astype(jnp.int32)
    p1 = jnp.sum(base * oh1, axis=-1, keepdims=True).astype(jnp.int32)
    d0_ref[...] = jnp.where(p0 < C, i0 * C + p0, SENT)
    d1_ref[...] = jnp.where(p1 < C, i1 * C + p1, SENT)


def _k1_call(x0f, ln0_scale, ln0_bias, Wr, br):
    f32 = jnp.float32
    return pl.pallas_call(
        _k1_body,
        grid=(NB,),
        in_specs=[
            pl.BlockSpec((BLK, D), lambda b: (b, 0)),
            pl.BlockSpec((1, D), lambda b: (0, 0)),
            pl.BlockSpec((1, D), lambda b: (0, 0)),
            pl.BlockSpec((D, E), lambda b: (0, 0)),
            pl.BlockSpec((1, E), lambda b: (0, 0)),
        ],
        out_specs=[
            pl.BlockSpec((BLK, D),
                         lambda b: (jnp.minimum(b, NSC // BLK - 1), 0)),
            pl.BlockSpec((BLK, D),
                         lambda b: (jnp.maximum(b - NSC // BLK, 0), 0)),
            pl.BlockSpec((BLK, 1), lambda b: (b, 0)),
            pl.BlockSpec((BLK, 1), lambda b: (b, 0)),
            pl.BlockSpec((BLK, 1), lambda b: (b, 0)),
            pl.BlockSpec((BLK, 1), lambda b: (b, 0)),
        ],
        out_shape=[
            jax.ShapeDtypeStruct((NSC, D), f32),
            jax.ShapeDtypeStruct((N - NSC, D), jnp.bfloat16),
            jax.ShapeDtypeStruct((N, 1), jnp.int32),
            jax.ShapeDtypeStruct((N, 1), jnp.int32),
            jax.ShapeDtypeStruct((N, 1), f32),
            jax.ShapeDtypeStruct((N, 1), f32),
        ],
        scratch_shapes=[pltpu.VMEM((1, E), f32)],
    )(x0f, ln0_scale.reshape(1, D), ln0_bias.reshape(1, D),
      Wr, br.reshape(1, E))


def _k1g_body(d0_ref, d1_ref, g0_ref, g1_ref, gsl_ref, osl_ref):
    e = pl.program_id(0)
    slot = e * C + lax.broadcasted_iota(jnp.int32, (C, 1), 0)
    eq0 = slot == d0_ref[...]
    eq1 = slot == d1_ref[...]
    s0 = jnp.sum(jnp.where(eq0, g0_ref[...], 0.0), axis=1, keepdims=True)
    s1 = jnp.sum(jnp.where(eq1, g1_ref[...], 0.0), axis=1, keepdims=True)
    gsl_ref[...] = s0 + s1
    low = lax.broadcasted_iota(jnp.int32, (1, N), 1) < NSC
    own = (eq0 | eq1) & low
    osl_ref[...] = jnp.sum(own.astype(jnp.float32), axis=1, keepdims=True)


def _k1g_call(d0r, d1r, g0r, g1r):
    row = pl.BlockSpec((1, N), lambda e: (0, 0))
    return pl.pallas_call(
        _k1g_body,
        grid=(E,),
        in_specs=[row, row, row, row],
        out_specs=[pl.BlockSpec((C, 1), lambda e: (e, 0)),
                   pl.BlockSpec((C, 1), lambda e: (e, 0))],
        out_shape=[jax.ShapeDtypeStruct((ROWS, 1), jnp.float32),
                   jax.ShapeDtypeStruct((ROWS, 1), jnp.float32)],
    )(d0r, d1r, g0r, g1r)


def _k2_body(x_hbm, d0_hbm, d1_hbm, xe_hbm, idx0_v, idx1_v, rows_v, sem):
    wid = lax.axis_index("s") * NC + lax.axis_index("c")
    base = wid * TPW4
    pltpu.sync_copy(d0_hbm.at[pl.ds(base, TPW4)], idx0_v)
    pltpu.sync_copy(d1_hbm.at[pl.ds(base, TPW4)], idx1_v)
    pltpu.sync_copy(x_hbm.at[pl.ds(base, TPW4)], rows_v)
    pltpu.async_copy(rows_v, xe_hbm.at[idx0_v], sem).wait()
    pltpu.async_copy(rows_v, xe_hbm.at[idx1_v], sem).wait()


@functools.cache
def _k2_kernel():
    return pl.kernel(
        _k2_body,
        mesh=plsc.VectorSubcoreMesh(core_axis_name="c", subcore_axis_name="s"),
        out_type=jax.ShapeDtypeStruct((ROWS, D), jnp.float32),
        scratch_types=[
            pltpu.VMEM((TPW4,), jnp.int32),
            pltpu.VMEM((TPW4,), jnp.int32),
            pltpu.VMEM((TPW4, D), jnp.float32),
            pltpu.SemaphoreType.DMA,
        ],
    )


def _k2_call(xact, d0f, d1f):
    return _k2_kernel()(xact, d0f, d1f)


def _k3_body(xe_ref, gsl_ref, osl_ref, d0_ref, d1_ref, xup_ref,
             w1_ref, b1_ref, s1_ref, bb1_ref, w2_ref, b2_ref,
             y_ref, yb_ref):
    e = pl.program_id(0)

    @pl.when(e < E)
    def _():
        slot = e * C + lax.broadcasted_iota(jnp.int32, (C, 1), 0)
        d0u = d0_ref[...][:, NSC:]
        d1u = d1_ref[...][:, NSC:]
        comb = ((slot == d0u) | (slot == d1u)).astype(jnp.bfloat16)
        mix = jnp.dot(comb, xup_ref[...], preferred_element_type=jnp.float32)
        x = jnp.where(osl_ref[...] > 0, xe_ref[...], mix)
        x = x.astype(jnp.bfloat16)
        h = jnp.dot(x, w1_ref[0], preferred_element_type=jnp.float32)
        h = h + b1_ref[0]
        mean = jnp.mean(h, axis=-1, keepdims=True)
        var = jnp.mean((h - mean) ** 2, axis=-1, keepdims=True)
        h = (h - mean) * lax.rsqrt(var + 1e-6) * s1_ref[0] + bb1_ref[0]
        h = jnp.maximum(h, 0.0).astype(jnp.bfloat16)
        y = jnp.dot(h, w2_ref[0], preferred_element_type=jnp.float32)
        g = gsl_ref[...]
        y = (y + b2_ref[0]) * g
        y_ref[...] = y
        yb_ref[...] = jnp.where(g > 0, y, 0.0).astype(jnp.bfloat16)

    @pl.when(e == E)
    def _():
        y_ref[...] = jnp.zeros((C, D), jnp.float32)
        yb_ref[...] = jnp.zeros((C, D), jnp.bfloat16)


def _k3_call(xe, gsl, osl, d0r, d1r, xup, W1b, b1, ln1_scale, ln1_bias,
             W2b, b2):
    vec = pl.BlockSpec((1, 1, H), lambda e: (jnp.minimum(e, E - 1), 0, 0))
    return pl.pallas_call(
        _k3_body,
        grid=(E + 1,),
        in_specs=[
            pl.BlockSpec((C, D), lambda e: (e, 0)),
            pl.BlockSpec((C, 1), lambda e: (e, 0)),
            pl.BlockSpec((C, 1), lambda e: (e, 0)),
            pl.BlockSpec((1, N), lambda e: (0, 0)),
            pl.BlockSpec((1, N), lambda e: (0, 0)),
            pl.BlockSpec((N - NSC, D), lambda e: (0, 0)),
            pl.BlockSpec((1, D, H), lambda e: (jnp.minimum(e, E - 1), 0, 0)),
            vec, vec, vec,
            pl.BlockSpec((1, H, D), lambda e: (jnp.minimum(e, E - 1), 0, 0)),
            pl.BlockSpec((1, 1, D), lambda e: (jnp.minimum(e, E - 1), 0, 0)),
        ],
        out_specs=[pl.BlockSpec((C, D), lambda e: (e, 0)),
                   pl.BlockSpec((C, D), lambda e: (e, 0))],
        out_shape=[jax.ShapeDtypeStruct((ROWS, D), jnp.float32),
                   jax.ShapeDtypeStruct((ROWS, D), jnp.bfloat16)],
    )(xe, gsl, osl, d0r, d1r, xup, W1b, b1.reshape(E, 1, H),
      ln1_scale.reshape(E, 1, H), ln1_bias.reshape(E, 1, H), W2b,
      b2.reshape(E, 1, D))


def _k4_body(y_hbm, d0_hbm, d1_hbm, ys_hbm, i0_v, i1_v, r0_v, r1_v, sem):
    wid = lax.axis_index("s") * NC + lax.axis_index("c")
    base = wid * TPW4
    pltpu.sync_copy(d0_hbm.at[pl.ds(base, TPW4)], i0_v)
    pltpu.sync_copy(d1_hbm.at[pl.ds(base, TPW4)], i1_v)
    pltpu.async_copy(y_hbm.at[i0_v], r0_v, sem).wait()
    pltpu.async_copy(y_hbm.at[i1_v], r1_v, sem).wait()

    def tok(i, _):
        for j in range(D // 16):
            sl = pl.ds(j * 16, 16)
            r0_v[i, sl] = r0_v[i, sl] + r1_v[i, sl]
        return 0

    lax.fori_loop(0, TPW4, tok, 0)
    pltpu.sync_copy(r0_v, ys_hbm.at[pl.ds(base, TPW4)])


@functools.cache
def _k4_kernel():
    return pl.kernel(
        _k4_body,
        mesh=plsc.VectorSubcoreMesh(core_axis_name="c", subcore_axis_name="s"),
        out_type=jax.ShapeDtypeStruct((NSC, D), jnp.float32),
        scratch_types=[
            pltpu.VMEM((TPW4,), jnp.int32),
            pltpu.VMEM((TPW4,), jnp.int32),
            pltpu.VMEM((TPW4, D), jnp.float32),
            pltpu.VMEM((TPW4, D), jnp.float32),
            pltpu.SemaphoreType.DMA,
        ],
    )


def _k4_call(ybuf, d0f, d1f):
    return _k4_kernel()(ybuf, d0f, d1f)


def _k4t_body(d0_ref, d1_ref, x0_ref, yb_ref, o_ref):
    slot = lax.broadcasted_iota(jnp.int32, (BLK4, ROWS), 1)
    comb = ((slot == d0_ref[...]) | (slot == d1_ref[...])).astype(jnp.bfloat16)
    mix = jnp.dot(comb, yb_ref[...], preferred_element_type=jnp.float32)
    o_ref[...] = x0_ref[...] + mix


def _k4t_call(d0, d1, x0f, ybf):
    off = NSC // BLK4
    col = pl.BlockSpec((BLK4, 1), lambda b: (off + b, 0))
    return pl.pallas_call(
        _k4t_body,
        grid=((N - NSC) // BLK4,),
        in_specs=[
            col, col,
            pl.BlockSpec((BLK4, D), lambda b: (off + b, 0)),
            pl.BlockSpec((ROWS, D), lambda b: (0, 0)),
        ],
        out_specs=pl.BlockSpec((BLK4, D), lambda b: (b, 0)),
        out_shape=jax.ShapeDtypeStruct((N - NSC, D), jnp.float32),
    )(d0, d1, x0f, ybf)


def _k5_body(x0_ref, ys_ref, yt_ref, o_ref):
    b = pl.program_id(0)

    @pl.when(b < NSC // BLK)
    def _():
        o_ref[...] = x0_ref[...] + ys_ref[...]

    @pl.when(b >= NSC // BLK)
    def _():
        o_ref[...] = yt_ref[...]


def _k5_call(x0f, ysum, ytc):
    nlo = NSC // BLK
    return pl.pallas_call(
        _k5_body,
        grid=(NB,),
        in_specs=[
            pl.BlockSpec((BLK, D), lambda b: (b, 0)),
            pl.BlockSpec((BLK, D), lambda b: (jnp.minimum(b, nlo - 1), 0)),
            pl.BlockSpec((BLK, D), lambda b: (jnp.maximum(b - nlo, 0), 0)),
        ],
        out_specs=pl.BlockSpec((BLK, D), lambda b: (b, 0)),
        out_shape=jax.ShapeDtypeStruct((N, D), jnp.float32),
    )(x0f, ysum, ytc)


def kernel(x0, ln0_scale, ln0_bias, Wr, br, W1, b1, ln1_scale, ln1_bias, W2, b2):
    x0f = x0.reshape(N, D)
    xlo, xup, d0, d1, g0, g1 = _k1_call(x0f, ln0_scale, ln0_bias, Wr, br)
    d0f = d0.reshape(N)
    d1f = d1.reshape(N)
    d0r = d0.reshape(1, N)
    d1r = d1.reshape(1, N)
    gsl, osl = _k1g_call(d0r, d1r, g0.reshape(1, N), g1.reshape(1, N))
    xe = _k2_call(xlo, d0f, d1f)
    ybuf, ybf = _k3_call(xe, gsl, osl, d0r, d1r, xup,
                         W1.astype(jnp.bfloat16), b1, ln1_scale,
                         ln1_bias, W2.astype(jnp.bfloat16), b2)
    ysum = _k4_call(ybuf, d0f, d1f)
    ytc = _k4t_call(d0, d1, x0f, ybf)
    out = _k5_call(x0f, ysum, ytc)
    return out.reshape(x0.shape)
